# baseline (speedup 1.0000x reference)
import jax
import jax.numpy as jnp
from jax import lax
from jax.experimental import pallas as pl
from jax.experimental.pallas import tpu as pltpu
from jax.experimental.pallas import tpu_sc as plsc

N = 50000
D = 256
G = 1024
GH = 2 * D

ABLK = 1024
NAB = 49
NP = ABLK * NAB

SEG_PER_TILE = G // 32
ACC_W = SEG_PER_TILE * D
ACC_C = SEG_PER_TILE * 16
CHUNK = 128


def _stage_a_body(hv_ref, wg_ref, bg_ref, w_ref, g_ref):
    i = pl.program_id(0)
    hv = hv_ref[...]
    wg = wg_ref[...]
    logits = jnp.sum(hv * wg, axis=1, keepdims=True) + bg_ref[0, 0]
    gate = 1.0 / (1.0 + jnp.exp(-logits))
    rows = lax.broadcasted_iota(jnp.int32, (ABLK, 1), 0) + i * ABLK
    valid = rows < N
    w_ref[...] = jnp.where(valid, gate * hv, 0.0)
    g_ref[...] = jnp.where(valid, jnp.broadcast_to(gate, (ABLK, 16)), 0.0)


def _bcast_lane(vec, lane):
    idx = jnp.full((16, 1), lane, jnp.int32)
    return lax.gather(
        vec, idx,
        dimension_numbers=lax.GatherDimensionNumbers(
            offset_dims=(), collapsed_slice_dims=(0,), start_index_map=(0,)),
        slice_sizes=(1,),
        mode=lax.GatherScatterMode.PROMISE_IN_BOUNDS)


def _sc_body(w_hbm, g_hbm, ids_hbm, bnd_hbm, zw_hbm, zc_hbm, pw_hbm, pc_hbm,
             wbuf, gbuf, idsv, bndv, acc, cacc):
    cid = lax.axis_index("c")
    sid = lax.axis_index("s")
    t = sid * 2 + cid
    iota16 = lax.iota(jnp.int32, 16)

    pltpu.sync_copy(zw_hbm, acc)
    pltpu.sync_copy(zc_hbm, cacc)
    pltpu.sync_copy(bnd_hbm, bndv)

    def extract(idx):
        tot = jnp.float32(0.0)
        for j in range(3):
            v = bndv[pl.ds(j * 16, 16)]
            tot += jnp.sum(jnp.where(iota16 + j * 16 == idx, v, 0.0))
        return tot.astype(jnp.int32)

    lo = extract(t)
    hi = extract(t + 1)
    seg0 = t * SEG_PER_TILE

    def chunk(q, carry):
        off = q * CHUNK
        pltpu.sync_copy(ids_hbm.at[pl.ds(off, CHUNK)], idsv)
        pltpu.sync_copy(w_hbm.at[pl.ds(off, CHUNK)], wbuf)
        pltpu.sync_copy(g_hbm.at[pl.ds(off, CHUNK)], gbuf)
        jlo = jnp.maximum(lo - off, 0)
        jhi = jnp.minimum(hi - off, CHUNK)

        def group(m, c2):
            vec = idsv[pl.ds(m * 16, 16)]
            for lane in range(16):
                jj = m * 16 + lane
                ok = (jj >= jlo) & (jj < jhi)
                mask = jnp.broadcast_to(ok, (16,))
                bc = _bcast_lane(vec, lane)
                loc = bc - seg0
                for k in range(16):
                    vals = wbuf[jj, pl.ds(k * 16, 16)]
                    plsc.addupdate_scatter(
                        acc, [loc * D + (k * 16) + iota16], vals, mask=mask)
                plsc.addupdate_scatter(
                    cacc, [loc * 16 + iota16], gbuf[jj], mask=mask)
            return c2

        lax.fori_loop(jlo // 16, (jhi + 15) // 16, group, 0)
        return carry

    lax.fori_loop(lo // CHUNK, (hi + CHUNK - 1) // CHUNK, chunk, 0)

    pltpu.sync_copy(acc, pw_hbm.at[pl.ds(t * ACC_W, ACC_W)])
    pltpu.sync_copy(cacc, pc_hbm.at[pl.ds(t * ACC_C, ACC_C)])


def _final_body(pw_ref, pc_ref, wp_ref, bp_ref, out_ref):
    s = pw_ref[...]
    c = pc_ref[:, 0:1]
    out_ref[...] = lax.dot_general(
        s, wp_ref[...], (((1,), (1,)), ((), ())),
        preferred_element_type=jnp.float32) + c * bp_ref[...]


def kernel(hv, segment_ids, W_gate, b_gate, W_proj, b_proj):
    bg = b_gate.reshape(1, 1)
    bp = b_proj.reshape(1, GH)
    ids = jnp.concatenate(
        [segment_ids.astype(jnp.int32),
         jnp.full((NP - N,), G - 1, jnp.int32)])
    bnd = jnp.searchsorted(
        ids, jnp.arange(33, dtype=jnp.int32) * SEG_PER_TILE).astype(jnp.float32)
    bnd48 = jnp.concatenate([bnd, jnp.full((15,), NP, jnp.float32)])
    zw = jnp.zeros((ACC_W,), jnp.float32)
    zc = jnp.zeros((ACC_C,), jnp.float32)

    w, g16 = pl.pallas_call(
        _stage_a_body,
        grid=(NAB,),
        in_specs=[
            pl.BlockSpec((ABLK, D), lambda i: (i, 0)),
            pl.BlockSpec((1, D), lambda i: (0, 0)),
            pl.BlockSpec((1, 1), lambda i: (0, 0)),
        ],
        out_specs=[
            pl.BlockSpec((ABLK, D), lambda i: (i, 0)),
            pl.BlockSpec((ABLK, 16), lambda i: (i, 0)),
        ],
        out_shape=[
            jax.ShapeDtypeStruct((NP, D), jnp.float32),
            jax.ShapeDtypeStruct((NP, 16), jnp.float32),
        ],
    )(hv, W_gate, bg)

    mesh = plsc.VectorSubcoreMesh(core_axis_name="c", subcore_axis_name="s")
    pw, pc = pl.kernel(
        _sc_body,
        out_type=[
            jax.ShapeDtypeStruct((G * D,), jnp.float32),
            jax.ShapeDtypeStruct((G * 16,), jnp.float32),
        ],
        mesh=mesh,
        compiler_params=pltpu.CompilerParams(needs_layout_passes=False),
        scratch_types=[
            pltpu.VMEM((CHUNK, D), jnp.float32),
            pltpu.VMEM((CHUNK, 16), jnp.float32),
            pltpu.VMEM((CHUNK,), jnp.int32),
            pltpu.VMEM((48,), jnp.float32),
            pltpu.VMEM((ACC_W,), jnp.float32),
            pltpu.VMEM((ACC_C,), jnp.float32),
        ],
    )(w, g16, ids, bnd48, zw, zc)

    out = pl.pallas_call(
        _final_body,
        grid=(1,),
        in_specs=[
            pl.BlockSpec((G, D), lambda i: (0, 0)),
            pl.BlockSpec((G, 16), lambda i: (0, 0)),
            pl.BlockSpec((GH, D), lambda i: (0, 0)),
            pl.BlockSpec((1, GH), lambda i: (0, 0)),
        ],
        out_specs=pl.BlockSpec((G, GH), lambda i: (0, 0)),
        out_shape=jax.ShapeDtypeStruct((G, GH), jnp.float32),
    )(pw.reshape(G, D), pc.reshape(G, 16), W_proj, bp)
    return out
